# baseline (speedup 1.0000x reference)
import functools
import math

import jax
import jax.numpy as jnp
from jax import lax
from jax.experimental import pallas as pl
from jax.experimental.pallas import tpu as pltpu
from jax.experimental.pallas import tpu_sc as plsc

_EPS = 1e-10
_RPAD = 24


def _make_sc_stats(B0, B, K, M, X, Y, R):
    NW = 32
    NB = B - B0
    WPB = NW // NB
    ROWS = X // WPB
    RC = 16
    NCH = ROWS // RC
    NV = Y // 16

    mesh = plsc.VectorSubcoreMesh(core_axis_name="c", subcore_axis_name="s")

    @functools.partial(
        pl.kernel,
        out_type=jax.ShapeDtypeStruct((NW, _RPAD, 16), jnp.float32),
        mesh=mesh,
        scratch_types=[
            pltpu.VMEM((2, K, RC, Y), jnp.float32),
            pltpu.VMEM((2, M, RC, Y), jnp.float32),
            pltpu.VMEM((2, 1, RC, Y), jnp.int32),
            pltpu.VMEM((_RPAD, 16), jnp.float32),
            pltpu.SemaphoreType.DMA,
            pltpu.SemaphoreType.DMA,
        ],
    )
    def sc_stats(pred_hbm, inp_hbm, heart_hbm, out_hbm,
                 pbuf, xbuf, hbuf, obuf, sem0, sem1):
        wid = lax.axis_index("s") * 2 + lax.axis_index("c")
        b = B0 + wid // WPB
        row0 = (wid % WPB) * ROWS
        sems = (sem0, sem1)

        def issue(ci, slot):
            r0 = row0 + ci * RC
            sem = sems[slot]
            return [
                pltpu.async_copy(
                    pred_hbm.at[b, :, pl.ds(r0, RC), :], pbuf.at[slot], sem),
                pltpu.async_copy(
                    inp_hbm.at[b, :, pl.ds(r0, RC), :], xbuf.at[slot], sem),
                pltpu.async_copy(
                    heart_hbm.at[b, :, pl.ds(r0, RC), :], hbuf.at[slot], sem),
            ]

        accs = tuple(jnp.zeros((16,), jnp.float32) for _ in range(R))
        pend = issue(0, 0)
        for ci in range(NCH):
            slot = ci % 2
            nxt = issue(ci + 1, 1 - slot) if ci + 1 < NCH else []
            for h in pend:
                h.wait()

            def body(i, carry, slot=slot):
                cs = list(carry)
                r = i // NV
                j = (i % NV) * 16
                hv = hbuf[slot, 0, r, pl.ds(j, 16)]
                mv = jnp.where(hv == 1, 1.0, 0.0).astype(jnp.float32)
                xv = [xbuf[slot, m, r, pl.ds(j, 16)] for m in range(M)]
                cs[0] = cs[0] + mv
                idx = 1
                for k in range(K):
                    pw = pbuf[slot, k, r, pl.ds(j, 16)] * mv
                    cs[idx] = cs[idx] + pw
                    for m in range(M):
                        cs[idx + 1 + m] = cs[idx + 1 + m] + pw * xv[m]
                    for m in range(M):
                        cs[idx + 1 + M + m] = (
                            cs[idx + 1 + M + m] + pw * (xv[m] * xv[m]))
                    idx += 1 + 2 * M
                return tuple(cs)

            accs = plsc.parallel_loop(
                0, RC * NV, carry=accs, unroll=4)(body)
            pend = nxt

        for r in range(R):
            obuf[r, :] = accs[r]
        for r in range(R, _RPAD):
            obuf[r, :] = jnp.zeros((16,), jnp.float32)
        pltpu.sync_copy(obuf, out_hbm.at[wid])

    return sc_stats


def _tc_stats_body(pred_ref, inp_ref, heart_ref, out_ref, acc_ref,
                   *, K, M, NC, S, Y, R):
    c = pl.program_id(1)
    mask = (heart_ref[0, 0] == 1).astype(jnp.float32)
    xs = [inp_ref[0, m] for m in range(M)]
    prods = [mask]
    for k in range(K):
        pw = pred_ref[0, k] * mask
        prods.append(pw)
        for m in range(M):
            prods.append(pw * xs[m])
        for m in range(M):
            prods.append(pw * (xs[m] * xs[m]))

    @pl.when(c == 0)
    def _():
        acc_ref[...] = jnp.zeros_like(acc_ref)

    for r, prod in enumerate(prods):
        acc_ref[r] += jnp.sum(prod.reshape(S // 8, 8, Y), axis=0)

    @pl.when(c == NC - 1)
    def _():
        for r in range(R):
            out_ref[0, r:r + 1, :] = jnp.sum(acc_ref[r], axis=0,
                                             keepdims=True)
        for r in range(R, _RPAD):
            out_ref[0, r:r + 1, :] = jnp.zeros((1, Y), jnp.float32)


def _loss_body(inp_ref, heart_ref, scp_ref, tcp_ref, out_ref, *, K, M, B):
    b = pl.program_id(0)
    c = pl.program_id(1)

    def tot(r):
        return (jnp.sum(scp_ref[0, :, r, :], keepdims=True)
                + jnp.sum(tcp_ref[0, r:r + 1, :], keepdims=True))

    n = tot(0)
    G = 2 * M + 1
    mus, hs, coefs = [], [], []
    for k in range(K):
        base_r = 1 + k * G
        sp = tot(base_r)
        d = sp + _EPS
        alpha = sp / n
        prodvar = None
        kmu, kh = [], []
        for m in range(M):
            t1 = tot(base_r + 1 + m)
            t2 = tot(base_r + 1 + M + m)
            mu = t1 / d
            var = jnp.maximum(t2 - 2.0 * mu * t1 + mu * mu * sp,
                              0.0) / d + _EPS
            kmu.append(mu)
            kh.append(-0.5 / var)
            prodvar = var if prodvar is None else prodvar * var
        mus.append(kmu)
        hs.append(kh)
        coefs.append(alpha * ((2.0 * math.pi) ** (-0.5 * M))
                     * jax.lax.rsqrt(prodvar))

    mask = (heart_ref[0, 0] == 1).astype(jnp.float32)
    xs = [inp_ref[0, m] for m in range(M)]
    s = None
    for k in range(K):
        e = None
        for m in range(M):
            dd = xs[m] - mus[k][m]
            t = hs[k][m] * (dd * dd)
            e = t if e is None else e + t
        term = coefs[k] * jnp.exp(e)
        s = term if s is None else s + term
    p = jnp.sum(mask * jnp.log(s + _EPS), keepdims=True)

    @pl.when((b == 0) & (c == 0))
    def _():
        out_ref[...] = jnp.zeros_like(out_ref)

    out_ref[...] += -(p / (n * B))


def kernel(predictions, inputs, heart):
    B, K, X, Y = predictions.shape
    M = inputs.shape[1]
    R = 1 + K * (1 + 2 * M)
    NW = 32
    B0 = B // 2
    WPB = NW // (B - B0)
    S = 256
    NC = X // S

    sc_stats = _make_sc_stats(B0, B, K, M, X, Y, R)
    scp = sc_stats(predictions, inputs, heart)
    scp = scp.reshape(B - B0, WPB, _RPAD, 16)
    scp = jnp.concatenate(
        [jnp.zeros((B0, WPB, _RPAD, 16), jnp.float32), scp], axis=0)

    tcp = pl.pallas_call(
        lambda pr, ir, hr, orf, ar: _tc_stats_body(
            pr, ir, hr, orf, ar, K=K, M=M, NC=NC, S=S, Y=Y, R=R),
        grid=(B0, NC),
        in_specs=[
            pl.BlockSpec((1, K, S, Y), lambda b, c: (b, 0, c, 0)),
            pl.BlockSpec((1, M, S, Y), lambda b, c: (b, 0, c, 0)),
            pl.BlockSpec((1, 1, S, Y), lambda b, c: (b, 0, c, 0)),
        ],
        out_specs=pl.BlockSpec((1, _RPAD, Y), lambda b, c: (b, 0, 0)),
        out_shape=jax.ShapeDtypeStruct((B0, _RPAD, Y), jnp.float32),
        scratch_shapes=[pltpu.VMEM((R, 8, Y), jnp.float32)],
    )(predictions[:B0], inputs[:B0], heart[:B0])
    tcp = jnp.concatenate(
        [tcp, jnp.zeros((B - B0, _RPAD, Y), jnp.float32)], axis=0)

    loss = pl.pallas_call(
        lambda ir, hr, sr, tr, orf: _loss_body(
            ir, hr, sr, tr, orf, K=K, M=M, B=B),
        grid=(B, NC),
        in_specs=[
            pl.BlockSpec((1, M, S, Y), lambda b, c: (b, 0, c, 0)),
            pl.BlockSpec((1, 1, S, Y), lambda b, c: (b, 0, c, 0)),
            pl.BlockSpec((1, WPB, _RPAD, 16), lambda b, c: (b, 0, 0, 0)),
            pl.BlockSpec((1, _RPAD, Y), lambda b, c: (b, 0, 0)),
        ],
        out_specs=pl.BlockSpec((1, 1), lambda b, c: (0, 0)),
        out_shape=jax.ShapeDtypeStruct((1, 1), jnp.float32),
    )(inputs, heart, scp, tcp)

    return loss.reshape(())


# speedup vs baseline: 1.2617x; 1.2617x over previous
import functools
import math

import jax
import jax.numpy as jnp
from jax import lax
from jax.experimental import pallas as pl
from jax.experimental.pallas import tpu as pltpu
from jax.experimental.pallas import tpu_sc as plsc

_EPS = 1e-10
_RPAD = 24


def _make_sc_stats(B0, B, K, M, X, Y, R):
    NW = 32
    NB = B - B0
    WPB = NW // NB
    ROWS = X // WPB
    RC = 16
    NCH = ROWS // RC
    NV = Y // 16

    mesh = plsc.VectorSubcoreMesh(core_axis_name="c", subcore_axis_name="s")

    @functools.partial(
        pl.kernel,
        out_type=jax.ShapeDtypeStruct((NW, _RPAD, 16), jnp.float32),
        mesh=mesh,
        scratch_types=[
            pltpu.VMEM((2, K, RC, Y), jnp.float32),
            pltpu.VMEM((2, M, RC, Y), jnp.float32),
            pltpu.VMEM((2, 1, RC, Y), jnp.int32),
            pltpu.VMEM((_RPAD, 16), jnp.float32),
            pltpu.SemaphoreType.DMA,
            pltpu.SemaphoreType.DMA,
        ],
    )
    def sc_stats(pred_hbm, inp_hbm, heart_hbm, out_hbm,
                 pbuf, xbuf, hbuf, obuf, sem0, sem1):
        wid = lax.axis_index("s") * 2 + lax.axis_index("c")
        b = B0 + wid // WPB
        row0 = (wid % WPB) * ROWS
        sems = (sem0, sem1)

        def issue(ci, slot):
            r0 = row0 + ci * RC
            sem = sems[slot]
            return [
                pltpu.async_copy(
                    pred_hbm.at[b, :, pl.ds(r0, RC), :], pbuf.at[slot], sem),
                pltpu.async_copy(
                    inp_hbm.at[b, :, pl.ds(r0, RC), :], xbuf.at[slot], sem),
                pltpu.async_copy(
                    heart_hbm.at[b, :, pl.ds(r0, RC), :], hbuf.at[slot], sem),
            ]

        accs = tuple(jnp.zeros((16,), jnp.float32) for _ in range(R))
        pend = issue(0, 0)
        for ci in range(NCH):
            slot = ci % 2
            nxt = issue(ci + 1, 1 - slot) if ci + 1 < NCH else []
            for h in pend:
                h.wait()

            def body(i, carry, slot=slot):
                cs = list(carry)
                r = i // NV
                j = (i % NV) * 16
                hv = hbuf[slot, 0, r, pl.ds(j, 16)]
                mv = jnp.where(hv == 1, 1.0, 0.0).astype(jnp.float32)
                xv = [xbuf[slot, m, r, pl.ds(j, 16)] for m in range(M)]
                cs[0] = cs[0] + mv
                idx = 1
                for k in range(K):
                    pw = pbuf[slot, k, r, pl.ds(j, 16)] * mv
                    cs[idx] = cs[idx] + pw
                    for m in range(M):
                        cs[idx + 1 + m] = cs[idx + 1 + m] + pw * xv[m]
                    for m in range(M):
                        cs[idx + 1 + M + m] = (
                            cs[idx + 1 + M + m] + pw * (xv[m] * xv[m]))
                    idx += 1 + 2 * M
                return tuple(cs)

            accs = plsc.parallel_loop(
                0, RC * NV, carry=accs, unroll=4)(body)
            pend = nxt

        for r in range(R):
            obuf[r, :] = accs[r]
        for r in range(R, _RPAD):
            obuf[r, :] = jnp.zeros((16,), jnp.float32)
        pltpu.sync_copy(obuf, out_hbm.at[wid])

    return sc_stats


def _tc_stats_body(pred_ref, inp_ref, heart_ref, out_ref, acc_ref,
                   *, K, M, NC, S, Y, R):
    c = pl.program_id(1)
    mask = (heart_ref[0, 0] == 1).astype(jnp.float32)
    xs = [inp_ref[0, m] for m in range(M)]
    prods = [mask]
    for k in range(K):
        pw = pred_ref[0, k] * mask
        prods.append(pw)
        for m in range(M):
            prods.append(pw * xs[m])
        for m in range(M):
            prods.append(pw * (xs[m] * xs[m]))

    @pl.when(c == 0)
    def _():
        acc_ref[...] = jnp.zeros_like(acc_ref)

    for r, prod in enumerate(prods):
        acc_ref[r] += jnp.sum(prod.reshape(S // 8, 8, Y), axis=0)

    @pl.when(c == NC - 1)
    def _():
        for r in range(R):
            out_ref[0, r:r + 1, :] = jnp.sum(acc_ref[r], axis=0,
                                             keepdims=True)
        for r in range(R, _RPAD):
            out_ref[0, r:r + 1, :] = jnp.zeros((1, Y), jnp.float32)


def _loss_body(inp_ref, heart_ref, scp_ref, tcp_ref, out_ref, *, K, M, B, B0):
    b = pl.program_id(0)
    c = pl.program_id(1)
    w_tc = jnp.where(b < B0, 1.0, 0.0)
    w_sc = 1.0 - w_tc

    def tot(r):
        return (w_sc * jnp.sum(scp_ref[0, :, r, :], keepdims=True)
                + w_tc * jnp.sum(tcp_ref[0, r:r + 1, :], keepdims=True))

    n = tot(0)
    G = 2 * M + 1
    mus, hs, coefs = [], [], []
    for k in range(K):
        base_r = 1 + k * G
        sp = tot(base_r)
        d = sp + _EPS
        alpha = sp / n
        prodvar = None
        kmu, kh = [], []
        for m in range(M):
            t1 = tot(base_r + 1 + m)
            t2 = tot(base_r + 1 + M + m)
            mu = t1 / d
            var = jnp.maximum(t2 - 2.0 * mu * t1 + mu * mu * sp,
                              0.0) / d + _EPS
            kmu.append(mu)
            kh.append(-0.5 / var)
            prodvar = var if prodvar is None else prodvar * var
        mus.append(kmu)
        hs.append(kh)
        coefs.append(alpha * ((2.0 * math.pi) ** (-0.5 * M))
                     * jax.lax.rsqrt(prodvar))

    mask = (heart_ref[0, 0] == 1).astype(jnp.float32)
    xs = [inp_ref[0, m] for m in range(M)]
    s = None
    for k in range(K):
        e = None
        for m in range(M):
            dd = xs[m] - mus[k][m]
            t = hs[k][m] * (dd * dd)
            e = t if e is None else e + t
        term = coefs[k] * jnp.exp(e)
        s = term if s is None else s + term
    p = jnp.sum(mask * jnp.log(s + _EPS), keepdims=True)

    @pl.when((b == 0) & (c == 0))
    def _():
        out_ref[...] = jnp.zeros_like(out_ref)

    out_ref[...] += -(p / (n * B))


def kernel(predictions, inputs, heart):
    B, K, X, Y = predictions.shape
    M = inputs.shape[1]
    R = 1 + K * (1 + 2 * M)
    NW = 32
    B0 = B // 2
    WPB = NW // (B - B0)
    S = 256
    NC = X // S

    sc_stats = _make_sc_stats(B0, B, K, M, X, Y, R)
    scp = sc_stats(predictions, inputs, heart)
    scp = scp.reshape(B - B0, WPB, _RPAD, 16)

    tcp = pl.pallas_call(
        lambda pr, ir, hr, orf, ar: _tc_stats_body(
            pr, ir, hr, orf, ar, K=K, M=M, NC=NC, S=S, Y=Y, R=R),
        grid=(B0, NC),
        in_specs=[
            pl.BlockSpec((1, K, S, Y), lambda b, c: (b, 0, c, 0)),
            pl.BlockSpec((1, M, S, Y), lambda b, c: (b, 0, c, 0)),
            pl.BlockSpec((1, 1, S, Y), lambda b, c: (b, 0, c, 0)),
        ],
        out_specs=pl.BlockSpec((1, _RPAD, Y), lambda b, c: (b, 0, 0)),
        out_shape=jax.ShapeDtypeStruct((B0, _RPAD, Y), jnp.float32),
        scratch_shapes=[pltpu.VMEM((R, 8, Y), jnp.float32)],
    )(predictions, inputs, heart)

    loss = pl.pallas_call(
        lambda ir, hr, sr, tr, orf: _loss_body(
            ir, hr, sr, tr, orf, K=K, M=M, B=B, B0=B0),
        grid=(B, NC),
        in_specs=[
            pl.BlockSpec((1, M, S, Y), lambda b, c: (b, 0, c, 0)),
            pl.BlockSpec((1, 1, S, Y), lambda b, c: (b, 0, c, 0)),
            pl.BlockSpec(
                (1, WPB, _RPAD, 16),
                lambda b, c: (jnp.maximum(b - B0, 0), 0, 0, 0)),
            pl.BlockSpec(
                (1, _RPAD, Y),
                lambda b, c: (jnp.minimum(b, B0 - 1), 0, 0)),
        ],
        out_specs=pl.BlockSpec((1, 1), lambda b, c: (0, 0)),
        out_shape=jax.ShapeDtypeStruct((1, 1), jnp.float32),
    )(inputs, heart, scp, tcp)

    return loss.reshape(())
